# Initial kernel scaffold; baseline (speedup 1.0000x reference)
import functools

import jax
import jax.numpy as jnp
from jax import lax
from jax.experimental import pallas as pl
from jax.experimental.pallas import tpu as pltpu
from jax.experimental.pallas import tpu_sc as plsc

NC = 2
NS = 16
NW = NC * NS
CH = 128
LANES = 16


def _matmul_body(x_ref, w_ref, h_ref):
    h_ref[...] = jnp.dot(x_ref[...], w_ref[...],
                         preferred_element_type=jnp.float32)


def _combine_body(p_ref, o_ref):
    o_ref[...] = jnp.maximum(p_ref[0] + p_ref[1], 0.0)


def _sc_edge_kernel(n_nodes, d, n_chunks_per_worker):
    rows_per_tile = n_nodes // NS
    mesh = plsc.VectorSubcoreMesh(core_axis_name="c", subcore_axis_name="s")

    @functools.partial(
        pl.kernel,
        out_type=jax.ShapeDtypeStruct((NC, n_nodes, d), jnp.float32),
        mesh=mesh,
        scratch_types=[
            pltpu.VMEM((CH,), jnp.int32),
            pltpu.VMEM((CH,), jnp.int32),
            pltpu.VMEM((CH,), jnp.float32),
            pltpu.VMEM((CH, d), jnp.float32),
            pltpu.VMEM_SHARED((n_nodes, d), jnp.float32),
            pltpu.SemaphoreType.DMA,
        ],
    )
    def body(h_hbm, src_hbm, dst_hbm, val_hbm, zeros_hbm, out_hbm,
             src_v, dst_v, val_v, rows_v, acc, sem):
        c = lax.axis_index("c")
        s = lax.axis_index("s")
        wid = s * NC + c

        pltpu.sync_copy(zeros_hbm.at[pl.ds(s * rows_per_tile, rows_per_tile)],
                        acc.at[pl.ds(s * rows_per_tile, rows_per_tile)])
        plsc.subcore_barrier()

        def chunk_body(k, carry):
            off = (wid + k * NW) * CH
            pltpu.sync_copy(src_hbm.at[pl.ds(off, CH)], src_v)
            pltpu.sync_copy(dst_hbm.at[pl.ds(off, CH)], dst_v)
            pltpu.sync_copy(val_hbm.at[pl.ds(off, CH)], val_v)
            pltpu.async_copy(h_hbm.at[src_v], rows_v, sem).wait()

            def scale_body(e, carry2):
                vb = plsc.load_gather(
                    val_v, [jnp.full((LANES,), e, dtype=jnp.int32)])
                for j in range(d // LANES):
                    sl = pl.ds(j * LANES, LANES)
                    rows_v[e, sl] = rows_v[e, sl] * vb
                return carry2

            lax.fori_loop(0, CH, scale_body, 0, unroll=2)

            pltpu.sync_copy(rows_v, acc.at[dst_v], add=True)
            return carry

        lax.fori_loop(0, n_chunks_per_worker, chunk_body, 0)
        plsc.subcore_barrier()

        pltpu.sync_copy(
            acc.at[pl.ds(s * rows_per_tile, rows_per_tile)],
            out_hbm.at[c, pl.ds(s * rows_per_tile, rows_per_tile)])

    return body


@jax.jit
def kernel(x, edge_index, adj_values, W):
    n, d_in = x.shape
    d_out = W.shape[1]
    e = adj_values.shape[0]

    h = pl.pallas_call(
        _matmul_body,
        grid=(n // 1000,),
        in_specs=[pl.BlockSpec((1000, d_in), lambda i: (i, 0)),
                  pl.BlockSpec((d_in, d_out), lambda i: (0, 0))],
        out_specs=pl.BlockSpec((1000, d_out), lambda i: (i, 0)),
        out_shape=jax.ShapeDtypeStruct((n, d_out), jnp.float32),
    )(x, W)

    block = NW * CH
    e_pad = ((e + block - 1) // block) * block
    pad = e_pad - e
    src = jnp.concatenate([edge_index[0], jnp.zeros((pad,), jnp.int32)])
    dst = jnp.concatenate([edge_index[1], jnp.zeros((pad,), jnp.int32)])
    val = jnp.concatenate([adj_values, jnp.zeros((pad,), jnp.float32)])
    zeros = jnp.zeros((n, d_out), jnp.float32)

    partial = _sc_edge_kernel(n, d_out, e_pad // block)(h, src, dst, val, zeros)

    out = pl.pallas_call(
        _combine_body,
        grid=(n // 1000,),
        in_specs=[pl.BlockSpec((NC, 1000, d_out), lambda i: (0, i, 0))],
        out_specs=pl.BlockSpec((1000, d_out), lambda i: (i, 0)),
        out_shape=jax.ShapeDtypeStruct((n, d_out), jnp.float32),
    )(partial)
    return out


# speedup vs baseline: 5.5733x; 5.5733x over previous
import functools

import jax
import jax.numpy as jnp
from jax import lax
from jax.experimental import pallas as pl
from jax.experimental.pallas import tpu as pltpu
from jax.experimental.pallas import tpu_sc as plsc

NC = 2
NS = 16
NW = NC * NS
CH = 128
LANES = 16


def _matmul_body(x_ref, w_ref, h_ref):
    h_ref[...] = jnp.dot(x_ref[...], w_ref[...],
                         preferred_element_type=jnp.float32)


def _combine_body(p_ref, o_ref):
    o_ref[...] = jnp.maximum(p_ref[0] + p_ref[1], 0.0)


def _sc_edge_kernel(n_pad, d, n_chunks):
    rows_per_tile = n_pad // NS
    mesh = plsc.VectorSubcoreMesh(core_axis_name="c", subcore_axis_name="s")

    @functools.partial(
        pl.kernel,
        out_type=jax.ShapeDtypeStruct((NC, n_pad, d), jnp.float32),
        mesh=mesh,
        scratch_types=[
            pltpu.VMEM((CH,), jnp.int32),
            pltpu.VMEM((CH,), jnp.int32),
            pltpu.VMEM((CH,), jnp.float32),
            pltpu.VMEM((CH, d), jnp.float32),
            pltpu.VMEM_SHARED((n_pad, d), jnp.float32),
            pltpu.SemaphoreType.DMA,
        ],
    )
    def body(h_hbm, ei_hbm, val_hbm, out_hbm,
             src_v, dst_v, val_v, rows_v, acc, sem):
        c = lax.axis_index("c")
        s = lax.axis_index("s")
        wid = s * NC + c

        zero16 = jnp.zeros((LANES,), jnp.float32)

        def zc(i, carry):
            for j in range(d // LANES):
                rows_v[i, pl.ds(j * LANES, LANES)] = zero16
            return carry

        lax.fori_loop(0, CH, zc, 0)
        for b in range(rows_per_tile // CH):
            pltpu.sync_copy(
                rows_v, acc.at[pl.ds(s * rows_per_tile + b * CH, CH)])
        plsc.subcore_barrier()

        n_my = (n_chunks - wid + NW - 1) // NW

        def chunk_body(k, carry):
            off = (wid + k * NW) * CH
            pltpu.sync_copy(ei_hbm.at[0, pl.ds(off, CH)], src_v)
            pltpu.sync_copy(ei_hbm.at[1, pl.ds(off, CH)], dst_v)
            pltpu.sync_copy(val_hbm.at[pl.ds(off, CH)], val_v)
            pltpu.async_copy(h_hbm.at[src_v], rows_v, sem).wait()

            def scale_body(g, carry2):
                v16 = val_v[pl.ds(g * LANES, LANES)]
                for l in range(LANES):
                    e = g * LANES + l
                    vb = v16[l]
                    for j in range(d // LANES):
                        sl = pl.ds(j * LANES, LANES)
                        rows_v[e, sl] = rows_v[e, sl] * vb
                return carry2

            lax.fori_loop(0, CH // LANES, scale_body, 0)

            pltpu.sync_copy(rows_v, acc.at[dst_v], add=True)
            return carry

        lax.fori_loop(0, n_my, chunk_body, 0)
        plsc.subcore_barrier()

        pltpu.sync_copy(
            acc.at[pl.ds(s * rows_per_tile, rows_per_tile)],
            out_hbm.at[c, pl.ds(s * rows_per_tile, rows_per_tile)])

    return body


@jax.jit
def kernel(x, edge_index, adj_values, W):
    n, d_in = x.shape
    d_out = W.shape[1]
    e = adj_values.shape[0]
    assert e % CH == 0

    n_pad = ((n + NS * CH - 1) // (NS * CH)) * (NS * CH)
    x_p = jnp.pad(x, ((0, n_pad - n), (0, 0)))

    h = pl.pallas_call(
        _matmul_body,
        grid=(n_pad // 1024,),
        in_specs=[pl.BlockSpec((1024, d_in), lambda i: (i, 0)),
                  pl.BlockSpec((d_in, d_out), lambda i: (0, 0))],
        out_specs=pl.BlockSpec((1024, d_out), lambda i: (i, 0)),
        out_shape=jax.ShapeDtypeStruct((n_pad, d_out), jnp.float32),
    )(x_p, W)

    partial = _sc_edge_kernel(n_pad, d_out, e // CH)(h, edge_index, adj_values)

    out = pl.pallas_call(
        _combine_body,
        grid=(n // 1000,),
        in_specs=[pl.BlockSpec((NC, 1000, d_out), lambda i: (0, i, 0))],
        out_specs=pl.BlockSpec((1000, d_out), lambda i: (i, 0)),
        out_shape=jax.ShapeDtypeStruct((n, d_out), jnp.float32),
    )(partial)
    return out


# speedup vs baseline: 9.6875x; 1.7382x over previous
import functools

import jax
import jax.numpy as jnp
from jax import lax
from jax.experimental import pallas as pl
from jax.experimental.pallas import tpu as pltpu
from jax.experimental.pallas import tpu_sc as plsc

NC = 2
NS = 16
NW = NC * NS
CH = 64
NB = 4
LANES = 16


def _proj_body(p_ref, w_ref, o_ref):
    agg = p_ref[0] + p_ref[1]
    o_ref[...] = jnp.maximum(
        jnp.dot(agg, w_ref[...], preferred_element_type=jnp.float32), 0.0)


def _sc_agg_kernel(n_pad, d, n_chunks):
    rows_per_tile = n_pad // NS
    n_full = n_chunks // NW
    n_rem = n_chunks % NW
    n_blocks = n_full // NB
    n_left = n_full % NB
    mesh = plsc.VectorSubcoreMesh(core_axis_name="c", subcore_axis_name="s")

    @functools.partial(
        pl.kernel,
        out_type=jax.ShapeDtypeStruct((NC, n_pad, d), jnp.float32),
        mesh=mesh,
        scratch_types=[
            [pltpu.VMEM((CH,), jnp.int32) for _ in range(NB)],
            [pltpu.VMEM((CH,), jnp.int32) for _ in range(NB)],
            [pltpu.VMEM((CH,), jnp.float32) for _ in range(NB)],
            [pltpu.VMEM((CH, d), jnp.float32) for _ in range(NB)],
            pltpu.VMEM_SHARED((n_pad, d), jnp.float32),
            [pltpu.SemaphoreType.DMA for _ in range(NB)],
            [pltpu.SemaphoreType.DMA for _ in range(NB)],
            [pltpu.SemaphoreType.DMA for _ in range(NB)],
            [pltpu.SemaphoreType.DMA for _ in range(NB)],
            [pltpu.SemaphoreType.DMA for _ in range(NB)],
        ],
    )
    def body(x_hbm, ei_hbm, val_hbm, out_hbm,
             src6, dst6, val6, rows6, acc, isem, dsem, vsem, gsem, ssem):
        c = lax.axis_index("c")
        s = lax.axis_index("s")
        wid = s * NC + c

        zero16 = jnp.zeros((LANES,), jnp.float32)

        def zc(i, carry):
            for j in range(d // LANES):
                rows6[0][i, pl.ds(j * LANES, LANES)] = zero16
            return carry

        lax.fori_loop(0, CH, zc, 0)
        for b in range(rows_per_tile // CH):
            pltpu.sync_copy(
                rows6[0], acc.at[pl.ds(s * rows_per_tile + b * CH, CH)])
        plsc.subcore_barrier()

        def scale(rows_v, val_v):
            def scale_body(g, carry):
                v16 = val_v[pl.ds(g * LANES, LANES)]
                for l in range(LANES):
                    e = g * LANES + l
                    vb = v16[l]
                    for j in range(d // LANES):
                        sl = pl.ds(j * LANES, LANES)
                        rows_v[e, sl] = rows_v[e, sl] * vb
                return carry
            lax.fori_loop(0, CH // LANES, scale_body, 0)

        def run_block(k0, nb):
            offs = [(wid + (k0 + b) * NW) * CH for b in range(nb)]
            hi = [pltpu.async_copy(ei_hbm.at[0, pl.ds(offs[b], CH)],
                                   src6[b], isem[b]) for b in range(nb)]
            hd = [pltpu.async_copy(ei_hbm.at[1, pl.ds(offs[b], CH)],
                                   dst6[b], dsem[b]) for b in range(nb)]
            hv = [pltpu.async_copy(val_hbm.at[pl.ds(offs[b], CH)],
                                   val6[b], vsem[b]) for b in range(nb)]
            hg = []
            for b in range(nb):
                hi[b].wait()
                hg.append(pltpu.async_copy(x_hbm.at[src6[b]],
                                           rows6[b], gsem[b]))
            hs = []
            for b in range(nb):
                hg[b].wait()
                hv[b].wait()
                scale(rows6[b], val6[b])
                hd[b].wait()
                hs.append(pltpu.async_copy(rows6[b], acc.at[dst6[b]],
                                           ssem[b], add=True))
            for b in range(nb):
                hs[b].wait()

        def block_body(k3, carry):
            run_block(k3 * NB, NB)
            return carry

        lax.fori_loop(0, n_blocks, block_body, 0)
        if n_left:
            run_block(n_blocks * NB, n_left)

        if n_rem:
            @pl.when(wid < n_rem)
            def _():
                run_block(n_full, 1)

        plsc.subcore_barrier()
        pltpu.sync_copy(
            acc.at[pl.ds(s * rows_per_tile, rows_per_tile)],
            out_hbm.at[c, pl.ds(s * rows_per_tile, rows_per_tile)])

    return body


@jax.jit
def kernel(x, edge_index, adj_values, W):
    n, d_in = x.shape
    d_out = W.shape[1]
    e = adj_values.shape[0]
    assert e % CH == 0 and d_in == d_out

    n_pad = ((n + NS * CH - 1) // (NS * CH)) * (NS * CH)

    partial = _sc_agg_kernel(n_pad, d_in, e // CH)(x, edge_index, adj_values)

    out = pl.pallas_call(
        _proj_body,
        grid=(n // 1000,),
        in_specs=[pl.BlockSpec((NC, 1000, d_in), lambda i: (0, i, 0)),
                  pl.BlockSpec((d_in, d_out), lambda i: (0, 0))],
        out_specs=pl.BlockSpec((1000, d_out), lambda i: (i, 0)),
        out_shape=jax.ShapeDtypeStruct((n, d_out), jnp.float32),
    )(partial, W)
    return out


# speedup vs baseline: 10.6312x; 1.0974x over previous
import functools

import jax
import jax.numpy as jnp
from jax import lax
from jax.experimental import pallas as pl
from jax.experimental.pallas import tpu as pltpu
from jax.experimental.pallas import tpu_sc as plsc

NC = 2
NS = 16
NW = NC * NS
CH = 64
NB = 4
LANES = 16


def _proj_body(p_ref, w_ref, o_ref):
    agg = p_ref[0] + p_ref[1]
    o_ref[...] = jnp.maximum(
        jnp.dot(agg, w_ref[...], preferred_element_type=jnp.float32), 0.0)


RR = 5
RS = 10


def _sc_agg_kernel(n_pad, d, n_chunks):
    rows_per_tile = n_pad // NS
    n_full = n_chunks // NW
    n_rem = n_chunks % NW
    n_bodies = n_full // RS
    mesh = plsc.VectorSubcoreMesh(core_axis_name="c", subcore_axis_name="s")

    @functools.partial(
        pl.kernel,
        out_type=jax.ShapeDtypeStruct((NC, n_pad, d), jnp.float32),
        mesh=mesh,
        scratch_types=[
            [pltpu.VMEM((CH,), jnp.int32) for _ in range(RS)],
            [pltpu.VMEM((CH,), jnp.int32) for _ in range(RS)],
            [pltpu.VMEM((CH,), jnp.float32) for _ in range(RS)],
            [pltpu.VMEM((CH, d), jnp.float32) for _ in range(RR)],
            pltpu.VMEM_SHARED((n_pad, d), jnp.float32),
            [pltpu.SemaphoreType.DMA for _ in range(RS)],
            [pltpu.SemaphoreType.DMA for _ in range(RR)],
            [pltpu.SemaphoreType.DMA for _ in range(RR)],
        ],
    )
    def body(x_hbm, ei_hbm, val_hbm, out_hbm,
             src6, dst6, val6, rows6, acc, isem, gsem, ssem):
        c = lax.axis_index("c")
        s = lax.axis_index("s")
        wid = s * NC + c

        zero16 = jnp.zeros((LANES,), jnp.float32)

        def zc(i, carry):
            for j in range(d // LANES):
                rows6[0][i, pl.ds(j * LANES, LANES)] = zero16
            return carry

        lax.fori_loop(0, CH, zc, 0)
        for b in range(rows_per_tile // CH):
            pltpu.sync_copy(
                rows6[0], acc.at[pl.ds(s * rows_per_tile + b * CH, CH)])
        plsc.subcore_barrier()

        def scale(rows_v, val_v):
            def scale_body(g, carry):
                v16 = val_v[pl.ds(g * LANES, LANES)]
                for l in range(LANES):
                    e = g * LANES + l
                    vb = v16[l]
                    for j in range(d // LANES):
                        sl = pl.ds(j * LANES, LANES)
                        rows_v[e, sl] = rows_v[e, sl] * vb
                return carry
            lax.fori_loop(0, CH // LANES, scale_body, 0)

        def off(k):
            return (wid + k * NW) * CH

        def pred(k):
            return (wid + k * NW) < n_chunks

        def run_block(k0, nb):
            hl = []
            for i in range(nb):
                hl.append((
                    pltpu.async_copy(ei_hbm.at[0, pl.ds(off(k0 + i), CH)],
                                     src6[i], isem[i]),
                    pltpu.async_copy(ei_hbm.at[1, pl.ds(off(k0 + i), CH)],
                                     dst6[i], isem[i]),
                    pltpu.async_copy(val_hbm.at[pl.ds(off(k0 + i), CH)],
                                     val6[i], isem[i])))
            hg, hs = {}, {}

            def fire_gather(i):
                for h in hl[i]:
                    h.wait()
                hg[i] = pltpu.async_copy(x_hbm.at[src6[i]], rows6[i % RR],
                                         gsem[i % RR])

            for i in range(min(2, nb)):
                fire_gather(i)
            for i in range(nb):
                if i >= 3:
                    hs[i - 3].wait()
                if i + 2 < nb:
                    fire_gather(i + 2)
                hg[i].wait()
                scale(rows6[i % RR], val6[i])
                hs[i] = pltpu.async_copy(rows6[i % RR], acc.at[dst6[i]],
                                         ssem[i % RR], add=True)
            for i in range(max(0, nb - 3), nb):
                hs[i].wait()

        def loop_body(j, carry):
            run_block(j * RS, RS)
            return carry

        lax.fori_loop(0, n_bodies, loop_body, 0)

        t0 = n_bodies * RS
        if n_full - t0:
            run_block(t0, n_full - t0)
        if n_rem:
            @pl.when(wid < n_rem)
            def _():
                run_block(n_full, 1)

        plsc.subcore_barrier()
        pltpu.sync_copy(
            acc.at[pl.ds(s * rows_per_tile, rows_per_tile)],
            out_hbm.at[c, pl.ds(s * rows_per_tile, rows_per_tile)])

    return body


@jax.jit
def kernel(x, edge_index, adj_values, W):
    n, d_in = x.shape
    d_out = W.shape[1]
    e = adj_values.shape[0]
    assert e % CH == 0 and d_in == d_out

    n_pad = ((n + NS * CH - 1) // (NS * CH)) * (NS * CH)

    partial = _sc_agg_kernel(n_pad, d_in, e // CH)(x, edge_index, adj_values)

    out = pl.pallas_call(
        _proj_body,
        grid=(n // 1000,),
        in_specs=[pl.BlockSpec((NC, 1000, d_in), lambda i: (0, i, 0)),
                  pl.BlockSpec((d_in, d_out), lambda i: (0, 0))],
        out_specs=pl.BlockSpec((1000, d_out), lambda i: (i, 0)),
        out_shape=jax.ShapeDtypeStruct((n, d_out), jnp.float32),
    )(partial, W)
    return out
